# baseline (speedup 1.0000x reference)
import functools

import jax
import jax.numpy as jnp
from jax import lax
from jax.experimental import pallas as pl
from jax.experimental.pallas import tpu as pltpu
from jax.experimental.pallas import tpu_sc as plsc

N = 50000
D = 256
HID = 128
NSEG = 512


R1B = 2048
GRID1 = (N + R1B - 1) // R1B


def _u_body(x_ref, W1_ref, b1_ref, W2_ref, b2_ref, u_ref):
    xb = x_ref[...]
    W2 = W2_ref[...]
    h = jnp.tanh(jnp.dot(xb, W1_ref[...], preferred_element_type=jnp.float32)
                 + b1_ref[...][None, :])
    s_t = jax.lax.dot_general(W2, h, (((0,), (1,)), ((), ())),
                              preferred_element_type=jnp.float32) + b2_ref[0]
    shift = jnp.sum(jnp.abs(W2)) + jnp.abs(b2_ref[0])
    u_ref[...] = jnp.exp(s_t - shift)


def _u_stage(x, W1, b1, W2, b2):
    return pl.pallas_call(
        _u_body,
        grid=(GRID1,),
        in_specs=[
            pl.BlockSpec((R1B, D), lambda i: (i, 0)),
            pl.BlockSpec((D, HID), lambda i: (0, 0)),
            pl.BlockSpec((HID,), lambda i: (0,)),
            pl.BlockSpec((HID, 1), lambda i: (0, 0)),
            pl.BlockSpec((1,), lambda i: (0,)),
        ],
        out_specs=pl.BlockSpec((1, R1B), lambda i: (0, i)),
        out_shape=jax.ShapeDtypeStruct((1, N), jnp.float32),
    )(x, W1, b1, W2, b2)


NRG = 16
GROUP = 3120
CHUNK = 80
NCH_MAX = 40
FH = 128
SEGT = NSEG // NRG
RSEG = 8

_mesh = plsc.VectorSubcoreMesh(core_axis_name="c", subcore_axis_name="s")


def _sc_body(x_hbm, b_hbm, u_hbm, out_hbm, part_hbm, spart_hbm,
             acc, sacc, xb0, xb1, ub0, ub1, bb0, bb1,
             rbuf, ssbuf, invbuf, outbuf,
             sx0, sx1, su0, su1, sb0, sb1, rsem):
    rg = lax.axis_index("s")
    fh = lax.axis_index("c")
    base0 = rg * GROUP
    colo = fh * FH
    nch = jnp.where(rg == NRG - 1, NCH_MAX, NCH_MAX - 1)

    lanes = lax.iota(jnp.int32, 16)
    zeros16 = jnp.zeros((16,), jnp.float32)

    def _zero(i, _):
        for k in range(FH // 16):
            acc[pl.ds(i * FH + k * 16, 16)] = zeros16
        sacc[pl.ds(i * 16, 16)] = zeros16
        return 0

    lax.fori_loop(0, NSEG, _zero, 0)

    xbufs = (xb0, xb1)
    ubufs = (ub0, ub1)
    bbufs = (bb0, bb1)
    sxs = (sx0, sx1)
    sus = (su0, su1)
    sbs = (sb0, sb1)

    def _issue(c, b):
        rbase = base0 + c * CHUNK
        pltpu.async_copy(x_hbm.at[pl.ds(rbase, CHUNK), pl.ds(colo, FH)],
                         xbufs[b], sxs[b])
        pltpu.async_copy(u_hbm.at[pl.ds(rbase, CHUNK)], ubufs[b], sus[b])
        pltpu.async_copy(b_hbm.at[pl.ds(rbase, CHUNK)], bbufs[b], sbs[b])

    def _wait(c, b):
        rbase = base0 + c * CHUNK
        pltpu.make_async_copy(x_hbm.at[pl.ds(rbase, CHUNK), pl.ds(colo, FH)],
                              xbufs[b], sxs[b]).wait()
        pltpu.make_async_copy(u_hbm.at[pl.ds(rbase, CHUNK)], ubufs[b],
                              sus[b]).wait()
        pltpu.make_async_copy(b_hbm.at[pl.ds(rbase, CHUNK)], bbufs[b],
                              sbs[b]).wait()

    _issue(0, 0)
    _issue(1, 1)

    def _chunk(c, b):
        @pl.when(c < nch)
        def _():
            _wait(c, b)

            def _row16(t, _):
                row16 = t * 16
                g_vec = bbufs[b][pl.ds(row16, 16)]
                u_vec = ubufs[b][pl.ds(row16, 16)]
                for rr in range(16):
                    rrv = jnp.full((16,), rr, jnp.int32)
                    gb = jnp.take_along_axis(g_vec, rrv, axis=0)
                    ub = jnp.take_along_axis(u_vec, rrv, axis=0)
                    gbase = gb * FH + lanes
                    xvs = [xbufs[b][row16 + rr, pl.ds(k * 16, 16)]
                           for k in range(FH // 16)]
                    vals = [xv * ub for xv in xvs]
                    for k in range(FH // 16):
                        plsc.addupdate_scatter(acc, [gbase + k * 16], vals[k])
                    plsc.addupdate_scatter(sacc, [gb * 16 + lanes], ub)
                return 0

            lax.fori_loop(0, CHUNK // 16, _row16, 0)

        @pl.when(c + 2 < nch)
        def _():
            _issue(c + 2, b)

    def _pair(j, _):
        _chunk(2 * j, 0)
        _chunk(2 * j + 1, 1)
        return 0

    lax.fori_loop(0, NCH_MAX // 2, _pair, 0)

    pltpu.sync_copy(acc, part_hbm.at[rg, fh])
    pltpu.sync_copy(sacc, spart_hbm.at[rg, fh])
    plsc.subcore_barrier()

    def _round(r, _):
        segbase = r * 128 + rg * RSEG
        for p in range(NRG):
            pltpu.async_copy(spart_hbm.at[p, fh, pl.ds(segbase * 16,
                                                       RSEG * 16)],
                             ssbuf.at[p], rsem)
        for p in range(NRG):
            pltpu.make_async_copy(spart_hbm.at[p, fh, pl.ds(segbase * 16,
                                                            RSEG * 16)],
                                  ssbuf.at[p], rsem).wait()
        for j in range(RSEG):
            tot = ssbuf[0, pl.ds(j * 16, 16)]
            for p in range(1, NRG):
                tot = tot + ssbuf[p, pl.ds(j * 16, 16)]
            invbuf[pl.ds(j * 16, 16)] = 1.0 / (tot + 1e-30)
        for p in range(NRG):
            pltpu.async_copy(part_hbm.at[p, fh, pl.ds(segbase * FH,
                                                      RSEG * FH)],
                             rbuf.at[p], rsem)
        for p in range(NRG):
            pltpu.make_async_copy(part_hbm.at[p, fh, pl.ds(segbase * FH,
                                                           RSEG * FH)],
                                  rbuf.at[p], rsem).wait()
        for gg in range(RSEG):
            inv16 = invbuf[pl.ds(gg * 16, 16)]
            for k in range(FH // 16):
                tot = rbuf[0, pl.ds(gg * FH + k * 16, 16)]
                for p in range(1, NRG):
                    tot = tot + rbuf[p, pl.ds(gg * FH + k * 16, 16)]
                outbuf[gg, pl.ds(k * 16, 16)] = tot * inv16
        pltpu.sync_copy(outbuf,
                        out_hbm.at[pl.ds(segbase, RSEG), pl.ds(colo, FH)])
        return 0

    lax.fori_loop(0, NSEG // 128, _round, 0)


@functools.partial(
    pl.kernel,
    mesh=_mesh,
    compiler_params=pltpu.CompilerParams(needs_layout_passes=False),
    out_type=(jax.ShapeDtypeStruct((NSEG, D), jnp.float32),
              jax.ShapeDtypeStruct((NRG, 2, NSEG * FH), jnp.float32),
              jax.ShapeDtypeStruct((NRG, 2, NSEG * 16), jnp.float32)),
    scratch_types=[
        pltpu.VMEM((NSEG * FH,), jnp.float32),
        pltpu.VMEM((NSEG * 16,), jnp.float32),
        pltpu.VMEM((CHUNK, FH), jnp.float32),
        pltpu.VMEM((CHUNK, FH), jnp.float32),
        pltpu.VMEM((CHUNK,), jnp.float32),
        pltpu.VMEM((CHUNK,), jnp.float32),
        pltpu.VMEM((CHUNK,), jnp.int32),
        pltpu.VMEM((CHUNK,), jnp.int32),
        pltpu.VMEM((NRG, RSEG * FH), jnp.float32),
        pltpu.VMEM((NRG, RSEG * 16), jnp.float32),
        pltpu.VMEM((RSEG * 16,), jnp.float32),
        pltpu.VMEM((RSEG, FH), jnp.float32),
        pltpu.SemaphoreType.DMA,
        pltpu.SemaphoreType.DMA,
        pltpu.SemaphoreType.DMA,
        pltpu.SemaphoreType.DMA,
        pltpu.SemaphoreType.DMA,
        pltpu.SemaphoreType.DMA,
        pltpu.SemaphoreType.DMA,
    ],
)
def _sc_stage(x_hbm, b_hbm, u_hbm, out_hbm, part_hbm, spart_hbm, *rest):
    _sc_body(x_hbm, b_hbm, u_hbm, out_hbm, part_hbm, spart_hbm, *rest)


def kernel(x, batch, W1, b1, W2, b2):
    batch = batch.astype(jnp.int32)
    u = _u_stage(x, W1, b1, W2, b2).reshape(N)
    out, _, _ = _sc_stage(x, batch, u)
    return out


# speedup vs baseline: 1.1263x; 1.1263x over previous
import functools

import jax
import jax.numpy as jnp
from jax import lax
from jax.experimental import pallas as pl
from jax.experimental.pallas import tpu as pltpu
from jax.experimental.pallas import tpu_sc as plsc

N = 50000
D = 256
HID = 128
NSEG = 512


R1B = 4096
GRID1 = (N + R1B - 1) // R1B


def _u_body(x_ref, W1_ref, b1_ref, W2_ref, b2_ref, u_ref):
    xb = x_ref[...]
    W2 = W2_ref[...]
    h = jnp.tanh(jnp.dot(xb, W1_ref[...], preferred_element_type=jnp.float32)
                 + b1_ref[...][None, :])
    s_t = jax.lax.dot_general(W2, h, (((0,), (1,)), ((), ())),
                              preferred_element_type=jnp.float32) + b2_ref[0]
    shift = jnp.sum(jnp.abs(W2)) + jnp.abs(b2_ref[0])
    u_ref[...] = jnp.exp(s_t - shift)


def _u_stage(x, W1, b1, W2, b2):
    return pl.pallas_call(
        _u_body,
        grid=(GRID1,),
        in_specs=[
            pl.BlockSpec((R1B, D), lambda i: (i, 0)),
            pl.BlockSpec((D, HID), lambda i: (0, 0)),
            pl.BlockSpec((HID,), lambda i: (0,)),
            pl.BlockSpec((HID, 1), lambda i: (0, 0)),
            pl.BlockSpec((1,), lambda i: (0,)),
        ],
        out_specs=pl.BlockSpec((1, R1B), lambda i: (0, i)),
        out_shape=jax.ShapeDtypeStruct((1, N), jnp.float32),
    )(x, W1, b1, W2, b2)


NRG = 16
GROUP = 3120
CHUNK = 80
NCH_MAX = 40
FH = 128
SEGT = NSEG // NRG
RSEG = 4

_mesh = plsc.VectorSubcoreMesh(core_axis_name="c", subcore_axis_name="s")


def _sc_body(x_hbm, b_hbm, u_hbm, out_hbm,
             acc, sacc, xb0, xb1, ub0, ub1, bb0, bb1,
             spacc, spsacc, rbuf, ssbuf, invbuf, outbuf,
             sx0, sx1, su0, su1, sb0, sb1, rsem):
    rg = lax.axis_index("s")
    fh = lax.axis_index("c")
    base0 = rg * GROUP
    colo = fh * FH
    nch = jnp.where(rg == NRG - 1, NCH_MAX, NCH_MAX - 1)

    lanes = lax.iota(jnp.int32, 16)
    zeros16 = jnp.zeros((16,), jnp.float32)

    def _zero(i, _):
        for k in range(FH // 16):
            acc[pl.ds(i * FH + k * 16, 16)] = zeros16
        sacc[pl.ds(i * 16, 16)] = zeros16
        return 0

    plsc.parallel_loop(0, NSEG, unroll=2)(lambda i: _zero(i, None))

    xbufs = (xb0, xb1)
    ubufs = (ub0, ub1)
    bbufs = (bb0, bb1)
    sxs = (sx0, sx1)
    sus = (su0, su1)
    sbs = (sb0, sb1)

    def _issue(c, b):
        rbase = base0 + c * CHUNK
        pltpu.async_copy(x_hbm.at[pl.ds(rbase, CHUNK), pl.ds(colo, FH)],
                         xbufs[b], sxs[b])
        pltpu.async_copy(u_hbm.at[pl.ds(rbase, CHUNK)], ubufs[b], sus[b])
        pltpu.async_copy(b_hbm.at[pl.ds(rbase, CHUNK)], bbufs[b], sbs[b])

    def _wait(c, b):
        rbase = base0 + c * CHUNK
        pltpu.make_async_copy(x_hbm.at[pl.ds(rbase, CHUNK), pl.ds(colo, FH)],
                              xbufs[b], sxs[b]).wait()
        pltpu.make_async_copy(u_hbm.at[pl.ds(rbase, CHUNK)], ubufs[b],
                              sus[b]).wait()
        pltpu.make_async_copy(b_hbm.at[pl.ds(rbase, CHUNK)], bbufs[b],
                              sbs[b]).wait()

    _issue(0, 0)
    _issue(1, 1)

    def _chunk(c, b):
        @pl.when(c < nch)
        def _():
            _wait(c, b)

            def _row16(t, _):
                row16 = t * 16
                g_vec = bbufs[b][pl.ds(row16, 16)]
                u_vec = ubufs[b][pl.ds(row16, 16)]
                for rr in range(16):
                    rrv = jnp.full((16,), rr, jnp.int32)
                    gb = jnp.take_along_axis(g_vec, rrv, axis=0)
                    ub = jnp.take_along_axis(u_vec, rrv, axis=0)
                    gbase = gb * FH + lanes
                    xvs = [xbufs[b][row16 + rr, pl.ds(k * 16, 16)]
                           for k in range(FH // 16)]
                    vals = [xv * ub for xv in xvs]
                    for k in range(FH // 16):
                        plsc.addupdate_scatter(acc, [gbase + k * 16], vals[k])
                    plsc.addupdate_scatter(sacc, [gb * 16 + lanes], ub)
                return 0

            lax.fori_loop(0, CHUNK // 16, _row16, 0)

        @pl.when(c + 2 < nch)
        def _():
            _issue(c + 2, b)

    def _pair(j, _):
        _chunk(2 * j, 0)
        _chunk(2 * j + 1, 1)
        return 0

    lax.fori_loop(0, NCH_MAX // 2, _pair, 0)

    QW = NSEG * FH // 8
    pltpu.sync_copy(sacc, spsacc.at[rg])
    plsc.subcore_barrier()

    def _round(r, _):
        pltpu.sync_copy(acc.at[pl.ds(r * QW, QW)], spacc.at[rg])
        plsc.subcore_barrier()
        soff = (r * 64 + rg * RSEG) * 16
        for p in range(NRG):
            pltpu.async_copy(spsacc.at[p, pl.ds(soff, RSEG * 16)],
                             ssbuf.at[p], rsem)
        for p in range(NRG):
            pltpu.make_async_copy(spsacc.at[p, pl.ds(soff, RSEG * 16)],
                                  ssbuf.at[p], rsem).wait()
        for j in range(RSEG):
            tot = ssbuf[0, pl.ds(j * 16, 16)]
            for p in range(1, NRG):
                tot = tot + ssbuf[p, pl.ds(j * 16, 16)]
            invbuf[pl.ds(j * 16, 16)] = 1.0 / (tot + 1e-30)
        woff = rg * (RSEG * FH)
        for p in range(NRG):
            pltpu.async_copy(spacc.at[p, pl.ds(woff, RSEG * FH)],
                             rbuf.at[p], rsem)
        for p in range(NRG):
            pltpu.make_async_copy(spacc.at[p, pl.ds(woff, RSEG * FH)],
                                  rbuf.at[p], rsem).wait()
        for gg in range(RSEG):
            inv16 = invbuf[pl.ds(gg * 16, 16)]
            for k in range(FH // 16):
                tot = rbuf[0, pl.ds(gg * FH + k * 16, 16)]
                for p in range(1, NRG):
                    tot = tot + rbuf[p, pl.ds(gg * FH + k * 16, 16)]
                outbuf[gg, pl.ds(k * 16, 16)] = tot * inv16
        pltpu.sync_copy(outbuf,
                        out_hbm.at[pl.ds(r * 64 + rg * RSEG, RSEG),
                                   pl.ds(colo, FH)])
        plsc.subcore_barrier()
        return 0

    lax.fori_loop(0, 8, _round, 0)


@functools.partial(
    pl.kernel,
    mesh=_mesh,
    compiler_params=pltpu.CompilerParams(needs_layout_passes=False),
    out_type=jax.ShapeDtypeStruct((NSEG, D), jnp.float32),
    scratch_types=[
        pltpu.VMEM((NSEG * FH,), jnp.float32),
        pltpu.VMEM((NSEG * 16,), jnp.float32),
        pltpu.VMEM((CHUNK, FH), jnp.float32),
        pltpu.VMEM((CHUNK, FH), jnp.float32),
        pltpu.VMEM((CHUNK,), jnp.float32),
        pltpu.VMEM((CHUNK,), jnp.float32),
        pltpu.VMEM((CHUNK,), jnp.int32),
        pltpu.VMEM((CHUNK,), jnp.int32),
        pltpu.VMEM_SHARED((NRG, NSEG * FH // 8), jnp.float32),
        pltpu.VMEM_SHARED((NRG, NSEG * 16), jnp.float32),
        pltpu.VMEM((NRG, RSEG * FH), jnp.float32),
        pltpu.VMEM((NRG, RSEG * 16), jnp.float32),
        pltpu.VMEM((RSEG * 16,), jnp.float32),
        pltpu.VMEM((RSEG, FH), jnp.float32),
        pltpu.SemaphoreType.DMA,
        pltpu.SemaphoreType.DMA,
        pltpu.SemaphoreType.DMA,
        pltpu.SemaphoreType.DMA,
        pltpu.SemaphoreType.DMA,
        pltpu.SemaphoreType.DMA,
        pltpu.SemaphoreType.DMA,
    ],
)
def _sc_stage(x_hbm, b_hbm, u_hbm, out_hbm, *rest):
    _sc_body(x_hbm, b_hbm, u_hbm, out_hbm, *rest)


def kernel(x, batch, W1, b1, W2, b2):
    batch = batch.astype(jnp.int32)
    u = _u_stage(x, W1, b1, W2, b2).reshape(N)
    return _sc_stage(x, batch, u)


# speedup vs baseline: 1.1488x; 1.0199x over previous
import functools

import jax
import jax.numpy as jnp
from jax import lax
from jax.experimental import pallas as pl
from jax.experimental.pallas import tpu as pltpu
from jax.experimental.pallas import tpu_sc as plsc

N = 50000
D = 256
HID = 128
NSEG = 512


R1B = 4096
GRID1 = (N + R1B - 1) // R1B


def _u_body(x_ref, W1_ref, b1_ref, W2_ref, b2_ref, u_ref):
    xb = x_ref[...]
    W2 = W2_ref[...]
    h = jnp.tanh(jnp.dot(xb, W1_ref[...], preferred_element_type=jnp.float32)
                 + b1_ref[...][None, :])
    s_t = jax.lax.dot_general(W2, h, (((0,), (1,)), ((), ())),
                              preferred_element_type=jnp.float32) + b2_ref[0]
    shift = jnp.sum(jnp.abs(W2)) + jnp.abs(b2_ref[0])
    u_ref[...] = jnp.exp(s_t - shift).reshape(R1B)


def _u_stage(x, W1, b1, W2, b2):
    return pl.pallas_call(
        _u_body,
        grid=(GRID1,),
        in_specs=[
            pl.BlockSpec((R1B, D), lambda i: (i, 0)),
            pl.BlockSpec((D, HID), lambda i: (0, 0)),
            pl.BlockSpec((HID,), lambda i: (0,)),
            pl.BlockSpec((HID, 1), lambda i: (0, 0)),
            pl.BlockSpec((1,), lambda i: (0,)),
        ],
        out_specs=pl.BlockSpec((R1B,), lambda i: (i,)),
        out_shape=jax.ShapeDtypeStruct((N,), jnp.float32),
    )(x, W1, b1, W2, b2)


NRG = 16
GROUP = 3120
CHUNK = 80
NCH_MAX = 40
FH = 128
SEGT = NSEG // NRG
RSEG = 4

_mesh = plsc.VectorSubcoreMesh(core_axis_name="c", subcore_axis_name="s")


def _sc_body(x_hbm, b_hbm, u_hbm, out_hbm,
             acc, sacc, xb0, xb1, ub0, ub1, bb0, bb1,
             spacc, spsacc, rbuf, ssbuf, invbuf, outbuf,
             sx0, sx1, su0, su1, sb0, sb1, rsem):
    rg = lax.axis_index("s")
    fh = lax.axis_index("c")
    base0 = rg * GROUP
    colo = fh * FH
    nch = jnp.where(rg == NRG - 1, NCH_MAX, NCH_MAX - 1)

    lanes = lax.iota(jnp.int32, 16)
    zeros16 = jnp.zeros((16,), jnp.float32)

    def _zero(i, _):
        for k in range(FH // 16):
            acc[pl.ds(i * FH + k * 16, 16)] = zeros16
        sacc[pl.ds(i * 16, 16)] = zeros16
        return 0

    plsc.parallel_loop(0, NSEG, unroll=2)(lambda i: _zero(i, None))

    xbufs = (xb0, xb1)
    ubufs = (ub0, ub1)
    bbufs = (bb0, bb1)
    sxs = (sx0, sx1)
    sus = (su0, su1)
    sbs = (sb0, sb1)

    def _issue(c, b):
        rbase = base0 + c * CHUNK
        pltpu.async_copy(x_hbm.at[pl.ds(rbase, CHUNK), pl.ds(colo, FH)],
                         xbufs[b], sxs[b])
        pltpu.async_copy(u_hbm.at[pl.ds(rbase, CHUNK)], ubufs[b], sus[b])
        pltpu.async_copy(b_hbm.at[pl.ds(rbase, CHUNK)], bbufs[b], sbs[b])

    def _wait(c, b):
        rbase = base0 + c * CHUNK
        pltpu.make_async_copy(x_hbm.at[pl.ds(rbase, CHUNK), pl.ds(colo, FH)],
                              xbufs[b], sxs[b]).wait()
        pltpu.make_async_copy(u_hbm.at[pl.ds(rbase, CHUNK)], ubufs[b],
                              sus[b]).wait()
        pltpu.make_async_copy(b_hbm.at[pl.ds(rbase, CHUNK)], bbufs[b],
                              sbs[b]).wait()

    _issue(0, 0)
    _issue(1, 1)

    def _chunk(c, b):
        @pl.when(c < nch)
        def _():
            _wait(c, b)

            def _row16(t, _):
                row16 = t * 16
                g_vec = bbufs[b][pl.ds(row16, 16)]
                u_vec = ubufs[b][pl.ds(row16, 16)]
                for rr in range(16):
                    rrv = jnp.full((16,), rr, jnp.int32)
                    gb = jnp.take_along_axis(g_vec, rrv, axis=0)
                    ub = jnp.take_along_axis(u_vec, rrv, axis=0)
                    gbase = gb * FH + lanes
                    xvs = [xbufs[b][row16 + rr, pl.ds(k * 16, 16)]
                           for k in range(FH // 16)]
                    vals = [xv * ub for xv in xvs]
                    for k in range(FH // 16):
                        plsc.addupdate_scatter(acc, [gbase + k * 16], vals[k])
                    plsc.addupdate_scatter(sacc, [gb * 16 + lanes], ub)
                return 0

            lax.fori_loop(0, CHUNK // 16, _row16, 0)

        @pl.when(c + 2 < nch)
        def _():
            _issue(c + 2, b)

    def _pair(j, _):
        _chunk(2 * j, 0)
        _chunk(2 * j + 1, 1)
        return 0

    lax.fori_loop(0, NCH_MAX // 2, _pair, 0)

    QW = NSEG * FH // 8
    pltpu.sync_copy(sacc, spsacc.at[rg])
    plsc.subcore_barrier()

    def _round(r, _):
        pltpu.sync_copy(acc.at[pl.ds(r * QW, QW)], spacc.at[rg])
        plsc.subcore_barrier()
        soff = (r * 64 + rg * RSEG) * 16
        for p in range(NRG):
            pltpu.async_copy(spsacc.at[p, pl.ds(soff, RSEG * 16)],
                             ssbuf.at[p], rsem)
        for p in range(NRG):
            pltpu.make_async_copy(spsacc.at[p, pl.ds(soff, RSEG * 16)],
                                  ssbuf.at[p], rsem).wait()
        for j in range(RSEG):
            tot = ssbuf[0, pl.ds(j * 16, 16)]
            for p in range(1, NRG):
                tot = tot + ssbuf[p, pl.ds(j * 16, 16)]
            invbuf[pl.ds(j * 16, 16)] = 1.0 / (tot + 1e-30)
        woff = rg * (RSEG * FH)
        for p in range(NRG):
            pltpu.async_copy(spacc.at[p, pl.ds(woff, RSEG * FH)],
                             rbuf.at[p], rsem)
        for p in range(NRG):
            pltpu.make_async_copy(spacc.at[p, pl.ds(woff, RSEG * FH)],
                                  rbuf.at[p], rsem).wait()
        for gg in range(RSEG):
            inv16 = invbuf[pl.ds(gg * 16, 16)]
            for k in range(FH // 16):
                tot = rbuf[0, pl.ds(gg * FH + k * 16, 16)]
                for p in range(1, NRG):
                    tot = tot + rbuf[p, pl.ds(gg * FH + k * 16, 16)]
                outbuf[gg, pl.ds(k * 16, 16)] = tot * inv16
        pltpu.sync_copy(outbuf,
                        out_hbm.at[pl.ds(r * 64 + rg * RSEG, RSEG),
                                   pl.ds(colo, FH)])
        plsc.subcore_barrier()
        return 0

    lax.fori_loop(0, 8, _round, 0)


@functools.partial(
    pl.kernel,
    mesh=_mesh,
    compiler_params=pltpu.CompilerParams(needs_layout_passes=False),
    out_type=jax.ShapeDtypeStruct((NSEG, D), jnp.float32),
    scratch_types=[
        pltpu.VMEM((NSEG * FH,), jnp.float32),
        pltpu.VMEM((NSEG * 16,), jnp.float32),
        pltpu.VMEM((CHUNK, FH), jnp.float32),
        pltpu.VMEM((CHUNK, FH), jnp.float32),
        pltpu.VMEM((CHUNK,), jnp.float32),
        pltpu.VMEM((CHUNK,), jnp.float32),
        pltpu.VMEM((CHUNK,), jnp.int32),
        pltpu.VMEM((CHUNK,), jnp.int32),
        pltpu.VMEM_SHARED((NRG, NSEG * FH // 8), jnp.float32),
        pltpu.VMEM_SHARED((NRG, NSEG * 16), jnp.float32),
        pltpu.VMEM((NRG, RSEG * FH), jnp.float32),
        pltpu.VMEM((NRG, RSEG * 16), jnp.float32),
        pltpu.VMEM((RSEG * 16,), jnp.float32),
        pltpu.VMEM((RSEG, FH), jnp.float32),
        pltpu.SemaphoreType.DMA,
        pltpu.SemaphoreType.DMA,
        pltpu.SemaphoreType.DMA,
        pltpu.SemaphoreType.DMA,
        pltpu.SemaphoreType.DMA,
        pltpu.SemaphoreType.DMA,
        pltpu.SemaphoreType.DMA,
    ],
)
def _sc_stage(x_hbm, b_hbm, u_hbm, out_hbm, *rest):
    _sc_body(x_hbm, b_hbm, u_hbm, out_hbm, *rest)


def kernel(x, batch, W1, b1, W2, b2):
    batch = batch.astype(jnp.int32)
    u = _u_stage(x, W1, b1, W2, b2)
    return _sc_stage(x, batch, u)
